# baseline (speedup 1.0000x reference)
import functools

import jax
import jax.numpy as jnp
from jax.experimental import pallas as pl
from jax.experimental.pallas import tpu as pltpu

B, N, L = 1, 32, 256
NODE_IN, HID, EDGE_IN, EHID, HEADS, NBLK = 64, 64, 128, 64, 4, 3
C = HID
HO = HID * HEADS
TI = 32
TJ = 128
NPROG = L // TI

_PREC = jax.lax.Precision.HIGHEST


def _dot(a, b, dims):
    return jax.lax.dot_general(a, b, (dims, ((), ())), precision=_PREC,
                               preferred_element_type=jnp.float32)


def _mm(a, b):
    return _dot(a, b, ((a.ndim - 1,), (0,)))


def _elu(x):
    return jnp.where(x > 0, x, jnp.exp(x) - 1.0)


def _ln_last(x, g, b, eps=1e-5):
    m = jnp.mean(x, axis=-1, keepdims=True)
    v = jnp.mean((x - m) ** 2, axis=-1, keepdims=True)
    return (x - m) * jax.lax.rsqrt(v + eps) * g + b


def _mega_kernel(idx_ref, idxc_ref, pair_ref, seq_ref, msa_ref,
                 nlg_ref, nlb_ref, wq_ref, bq_ref, wk_ref, bk_ref,
                 wxm_ref, wxs_ref, bx_ref,
                 elg_ref, elb_ref, wep_ref, wes_ref, bee_ref,
                 bwq_ref, bbq_ref, bwk_ref, bbk_ref, bwv_ref, bbv_ref,
                 bwe_ref, bbe_ref, bws_ref, bbs_ref, blg_ref, blb_ref,
                 bwl_ref, bbl_ref, wxyz_ref, bxyz_ref,
                 out_ref, pe_scr):
    i = pl.program_id(0)

    pn = _ln_last(pair_ref[...], elg_ref[...], elb_ref[...])
    pe_t = _dot(wep_ref[...], pn.reshape(TI * L, EDGE_IN),
                ((0,), (1,))).reshape(EHID, TI, L)
    sep = (idx_ref[...] - idxc_ref[...]).astype(jnp.float32)
    ss = jnp.clip(jnp.log(jnp.abs(sep) + 1.0), 0.0, 5.5) * jnp.sign(sep)
    pe_t = pe_t + ss[None, :, :] * wes_ref[...][:, :, None] \
        + bee_ref[...][:, :, None]
    pe_scr[:, pl.ds(i * TI, TI), :] = _elu(pe_t)

    @pl.when(i == NPROG - 1)
    def _tail():
        msa_n = _ln_last(msa_ref[...], nlg_ref[...], nlb_ref[...])
        q0 = _mm(msa_n[0], wq_ref[...]) + bq_ref[...]
        k0 = (_mm(msa_n.reshape(N * L, NODE_IN), wk_ref[...])
              + bk_ref[...]).reshape(N, L, NODE_IN)
        attn = jnp.sum((q0 * (1.0 / 8.0))[None, :, :] * k0, axis=2)
        w0 = jnp.exp(attn - jnp.max(attn, axis=0, keepdims=True))
        w0 = w0 / jnp.sum(w0, axis=0, keepdims=True)
        msa_w = jnp.sum(w0[:, :, None] * msa_n, axis=0)
        x = _elu(_mm(msa_w, wxm_ref[...]) + _mm(seq_ref[...], wxs_ref[...])
                 + bx_ref[...])

        for t in range(NBLK):
            we = bwe_ref[t]
            be = bbe_ref[t]
            kn = _mm(x, bwk_ref[t]) + bbk_ref[t]
            vn = _mm(x, bwv_ref[t]) + bbv_ref[t]
            cols = []
            for jc in range(L // TJ):
                j0 = jc * TJ
                xj = x[j0:j0 + TJ]
                q = _mm(xj, bwq_ref[t]) + bbq_ref[t]
                pe = pe_scr[:, :, j0:j0 + TJ]
                row = jax.lax.broadcasted_iota(jnp.int32, (L, TJ), 0)
                col = jax.lax.broadcasted_iota(jnp.int32, (L, TJ), 1) + j0
                diag = row == col
                aggs = []
                for h in range(HEADS):
                    sl = slice(h * C, (h + 1) * C)
                    q_h, k_h, v_h = q[:, sl], kn[:, sl], vn[:, sl]
                    we_h, be_h = we[:, sl], be[:, sl]
                    qk = _dot(k_h, q_h, ((1,), (1,)))
                    g = _dot(we_h, q_h, ((1,), (1,)))
                    ae = jnp.sum(pe * g[:, None, :], axis=0)
                    qbe = _dot(be_h, q_h, ((1,), (1,)))
                    logits = (qk + ae + qbe) * (1.0 / 8.0)
                    logits = jnp.where(diag, -1e30, logits)
                    m = jnp.max(logits, axis=0, keepdims=True)
                    w = jnp.exp(logits - m)
                    denom = _dot(w, jnp.ones((L, 1), jnp.float32),
                                 ((0,), (0,)))
                    num_v = _dot(w, v_h, ((0,), (0,)))
                    p_t = jnp.sum(pe * w[None, :, :], axis=1)
                    eterm = _dot(p_t, we_h, ((0,), (0,))) + denom * be_h
                    aggs.append((num_v + eterm) / (denom + 1e-16))
                agg = jnp.concatenate(aggs, axis=1)
                agg = agg + _mm(xj, bws_ref[t]) + bbs_ref[t]
                hh = _ln_last(agg, blg_ref[t], blb_ref[t])
                cols.append(_elu(_mm(hh, bwl_ref[t]) + bbl_ref[t] + xj))
            x = jnp.concatenate(cols, axis=0)

        out_ref[...] = _mm(x, wxyz_ref[...]) + bxyz_ref[...]


def _full(shape):
    return pl.BlockSpec(shape, lambda *_: tuple(0 for _ in shape))


def kernel(seq1hot, idx, msa, pair, ln_node_g, ln_node_b, ln_edge_g,
           ln_edge_b, Wq, bq, Wk, bk, Wx, bx, We, be, blk_Wq, blk_bq,
           blk_Wk, blk_bk, blk_Wv, blk_bv, blk_We, blk_be, blk_Ws, blk_bs,
           blk_ln_g, blk_ln_b, blk_Wl, blk_bl, Wxyz, bxyz):
    f32 = jnp.float32
    r2 = lambda a: a.reshape(1, -1).astype(f32)
    r3 = lambda a: a.reshape(NBLK, 1, -1).astype(f32)

    xyz = pl.pallas_call(
        _mega_kernel,
        grid=(NPROG,),
        in_specs=[
            _full((1, L)),
            pl.BlockSpec((TI, 1), lambda i: (i, 0)),
            pl.BlockSpec((TI, L, EDGE_IN), lambda i: (i, 0, 0)),
            _full((L, 21)), _full((N, L, NODE_IN)),
            _full((1, NODE_IN)), _full((1, NODE_IN)),
            _full((NODE_IN, NODE_IN)), _full((1, NODE_IN)),
            _full((NODE_IN, NODE_IN)), _full((1, NODE_IN)),
            _full((NODE_IN, HID)), _full((21, HID)), _full((1, HID)),
            _full((1, EDGE_IN)), _full((1, EDGE_IN)),
            _full((EDGE_IN, EHID)), _full((EHID, 1)), _full((EHID, 1)),
            _full((NBLK, HID, HO)), _full((NBLK, 1, HO)),
            _full((NBLK, HID, HO)), _full((NBLK, 1, HO)),
            _full((NBLK, HID, HO)), _full((NBLK, 1, HO)),
            _full((NBLK, EHID, HO)), _full((NBLK, 1, HO)),
            _full((NBLK, HID, HO)), _full((NBLK, 1, HO)),
            _full((NBLK, 1, HO)), _full((NBLK, 1, HO)),
            _full((NBLK, HO, HID)), _full((NBLK, 1, HID)),
            _full((HID, 9)), _full((1, 9)),
        ],
        out_specs=_full((L, 9)),
        out_shape=jax.ShapeDtypeStruct((L, 9), f32),
        scratch_shapes=[pltpu.VMEM((EHID, L, L), f32)],
    )(idx.reshape(1, L), idx.reshape(L, 1), pair.reshape(L, L, EDGE_IN),
      seq1hot.reshape(L, 21), msa.reshape(N, L, NODE_IN),
      r2(ln_node_g), r2(ln_node_b), Wq, r2(bq), Wk, r2(bk),
      Wx[:NODE_IN], Wx[NODE_IN:], r2(bx),
      r2(ln_edge_g), r2(ln_edge_b), We[:EDGE_IN],
      We[EDGE_IN].reshape(EHID, 1), be.reshape(EHID, 1),
      blk_Wq, r3(blk_bq), blk_Wk, r3(blk_bk), blk_Wv, r3(blk_bv),
      blk_We, r3(blk_be), blk_Ws, r3(blk_bs), r3(blk_ln_g), r3(blk_ln_b),
      blk_Wl, r3(blk_bl), Wxyz, r2(bxyz))
    return xyz.reshape(B, L, 3, 3)


# speedup vs baseline: 1.2572x; 1.2572x over previous
import functools

import jax
import jax.numpy as jnp
from jax.experimental import pallas as pl
from jax.experimental.pallas import tpu as pltpu

B, N, L = 1, 32, 256
NODE_IN, HID, EDGE_IN, EHID, HEADS, NBLK = 64, 64, 128, 64, 4, 3
C = HID
HO = HID * HEADS
TI = 32
TJ = 128
NPROG = L // TI

_PREC = jax.lax.Precision.DEFAULT


def _dot(a, b, dims):
    return jax.lax.dot_general(a, b, (dims, ((), ())), precision=_PREC,
                               preferred_element_type=jnp.float32)


def _mm(a, b):
    return _dot(a, b, ((a.ndim - 1,), (0,)))


def _elu(x):
    return jnp.where(x > 0, x, jnp.exp(x) - 1.0)


def _ln_last(x, g, b, eps=1e-5):
    m = jnp.mean(x, axis=-1, keepdims=True)
    v = jnp.mean((x - m) ** 2, axis=-1, keepdims=True)
    return (x - m) * jax.lax.rsqrt(v + eps) * g + b


def _mega_kernel(idx_ref, idxc_ref, pair_ref, seq_ref, msa_ref,
                 nlg_ref, nlb_ref, wq_ref, bq_ref, wk_ref, bk_ref,
                 wxm_ref, wxs_ref, bx_ref,
                 elg_ref, elb_ref, wep_ref, wes_ref, bee_ref,
                 bwq_ref, bbq_ref, bwk_ref, bbk_ref, bwv_ref, bbv_ref,
                 bwe_ref, bbe_ref, bws_ref, bbs_ref, blg_ref, blb_ref,
                 bwl_ref, bbl_ref, wxyz_ref, bxyz_ref,
                 out_ref, pe_scr):
    i = pl.program_id(0)

    pn = _ln_last(pair_ref[...], elg_ref[...], elb_ref[...])
    pe_t = _dot(wep_ref[...], pn.reshape(TI * L, EDGE_IN),
                ((0,), (1,))).reshape(EHID, TI, L)
    sep = (idx_ref[...] - idxc_ref[...]).astype(jnp.float32)
    ss = jnp.clip(jnp.log(jnp.abs(sep) + 1.0), 0.0, 5.5) * jnp.sign(sep)
    pe_t = pe_t + ss[None, :, :] * wes_ref[...][:, :, None] \
        + bee_ref[...][:, :, None]
    pe_scr[:, pl.ds(i * TI, TI), :] = _elu(pe_t)

    @pl.when(i == NPROG - 1)
    def _tail():
        msa_n = _ln_last(msa_ref[...], nlg_ref[...], nlb_ref[...])
        q0 = _mm(msa_n[0], wq_ref[...]) + bq_ref[...]
        k0 = (_mm(msa_n.reshape(N * L, NODE_IN), wk_ref[...])
              + bk_ref[...]).reshape(N, L, NODE_IN)
        attn = jnp.sum((q0 * (1.0 / 8.0))[None, :, :] * k0, axis=2)
        w0 = jnp.exp(attn - jnp.max(attn, axis=0, keepdims=True))
        w0 = w0 / jnp.sum(w0, axis=0, keepdims=True)
        msa_w = jnp.sum(w0[:, :, None] * msa_n, axis=0)
        x = _elu(_mm(msa_w, wxm_ref[...]) + _mm(seq_ref[...], wxs_ref[...])
                 + bx_ref[...])

        for t in range(NBLK):
            we = bwe_ref[t]
            be = bbe_ref[t]
            kn = _mm(x, bwk_ref[t]) + bbk_ref[t]
            vn = _mm(x, bwv_ref[t]) + bbv_ref[t]
            cols = []
            for jc in range(L // TJ):
                j0 = jc * TJ
                xj = x[j0:j0 + TJ]
                q = _mm(xj, bwq_ref[t]) + bbq_ref[t]
                pe = pe_scr[:, :, j0:j0 + TJ]
                row = jax.lax.broadcasted_iota(jnp.int32, (L, TJ), 0)
                col = jax.lax.broadcasted_iota(jnp.int32, (L, TJ), 1) + j0
                diag = row == col
                aggs = []
                for h in range(HEADS):
                    sl = slice(h * C, (h + 1) * C)
                    q_h, k_h, v_h = q[:, sl], kn[:, sl], vn[:, sl]
                    we_h, be_h = we[:, sl], be[:, sl]
                    qk = _dot(k_h, q_h, ((1,), (1,)))
                    g = _dot(we_h, q_h, ((1,), (1,)))
                    ae = jnp.sum(pe * g[:, None, :], axis=0)
                    qbe = _dot(be_h, q_h, ((1,), (1,)))
                    logits = (qk + ae + qbe) * (1.0 / 8.0)
                    logits = jnp.where(diag, -1e30, logits)
                    m = jnp.max(logits, axis=0, keepdims=True)
                    w = jnp.exp(logits - m)
                    denom = _dot(w, jnp.ones((L, 1), jnp.float32),
                                 ((0,), (0,)))
                    num_v = _dot(w, v_h, ((0,), (0,)))
                    p_t = jnp.sum(pe * w[None, :, :], axis=1)
                    eterm = _dot(p_t, we_h, ((0,), (0,))) + denom * be_h
                    aggs.append((num_v + eterm) / (denom + 1e-16))
                agg = jnp.concatenate(aggs, axis=1)
                agg = agg + _mm(xj, bws_ref[t]) + bbs_ref[t]
                hh = _ln_last(agg, blg_ref[t], blb_ref[t])
                cols.append(_elu(_mm(hh, bwl_ref[t]) + bbl_ref[t] + xj))
            x = jnp.concatenate(cols, axis=0)

        out_ref[...] = _mm(x, wxyz_ref[...]) + bxyz_ref[...]


def _full(shape):
    return pl.BlockSpec(shape, lambda *_: tuple(0 for _ in shape))


def kernel(seq1hot, idx, msa, pair, ln_node_g, ln_node_b, ln_edge_g,
           ln_edge_b, Wq, bq, Wk, bk, Wx, bx, We, be, blk_Wq, blk_bq,
           blk_Wk, blk_bk, blk_Wv, blk_bv, blk_We, blk_be, blk_Ws, blk_bs,
           blk_ln_g, blk_ln_b, blk_Wl, blk_bl, Wxyz, bxyz):
    f32 = jnp.float32
    r2 = lambda a: a.reshape(1, -1).astype(f32)
    r3 = lambda a: a.reshape(NBLK, 1, -1).astype(f32)

    xyz = pl.pallas_call(
        _mega_kernel,
        grid=(NPROG,),
        in_specs=[
            _full((1, L)),
            pl.BlockSpec((TI, 1), lambda i: (i, 0)),
            pl.BlockSpec((TI, L, EDGE_IN), lambda i: (i, 0, 0)),
            _full((L, 21)), _full((N, L, NODE_IN)),
            _full((1, NODE_IN)), _full((1, NODE_IN)),
            _full((NODE_IN, NODE_IN)), _full((1, NODE_IN)),
            _full((NODE_IN, NODE_IN)), _full((1, NODE_IN)),
            _full((NODE_IN, HID)), _full((21, HID)), _full((1, HID)),
            _full((1, EDGE_IN)), _full((1, EDGE_IN)),
            _full((EDGE_IN, EHID)), _full((EHID, 1)), _full((EHID, 1)),
            _full((NBLK, HID, HO)), _full((NBLK, 1, HO)),
            _full((NBLK, HID, HO)), _full((NBLK, 1, HO)),
            _full((NBLK, HID, HO)), _full((NBLK, 1, HO)),
            _full((NBLK, EHID, HO)), _full((NBLK, 1, HO)),
            _full((NBLK, HID, HO)), _full((NBLK, 1, HO)),
            _full((NBLK, 1, HO)), _full((NBLK, 1, HO)),
            _full((NBLK, HO, HID)), _full((NBLK, 1, HID)),
            _full((HID, 9)), _full((1, 9)),
        ],
        out_specs=_full((L, 9)),
        out_shape=jax.ShapeDtypeStruct((L, 9), f32),
        scratch_shapes=[pltpu.VMEM((EHID, L, L), f32)],
    )(idx.reshape(1, L), idx.reshape(L, 1), pair.reshape(L, L, EDGE_IN),
      seq1hot.reshape(L, 21), msa.reshape(N, L, NODE_IN),
      r2(ln_node_g), r2(ln_node_b), Wq, r2(bq), Wk, r2(bk),
      Wx[:NODE_IN], Wx[NODE_IN:], r2(bx),
      r2(ln_edge_g), r2(ln_edge_b), We[:EDGE_IN],
      We[EDGE_IN].reshape(EHID, 1), be.reshape(EHID, 1),
      blk_Wq, r3(blk_bq), blk_Wk, r3(blk_bk), blk_Wv, r3(blk_bv),
      blk_We, r3(blk_be), blk_Ws, r3(blk_bs), r3(blk_ln_g), r3(blk_ln_b),
      blk_Wl, r3(blk_bl), Wxyz, r2(bxyz))
    return xyz.reshape(B, L, 3, 3)
